# baseline (speedup 1.0000x reference)
import functools

import jax
import jax.numpy as jnp
from jax import lax
from jax.experimental import pallas as pl
from jax.experimental.pallas import tpu as pltpu
from jax.experimental.pallas import tpu_sc as plsc

N = 10000
D = 128
H = 256
E = 320000
BN_EPS = 1e-5

NC = 2
NS = 16
NT = NC * NS
B = 128
GT = 20
NG = 4
T = GT * NG
EP = NT * T * B
PAD = EP - E
NROWS = N + 112
NPADROWS = NROWS - N
RPT_Z = NROWS // NS
RPT_O = 624
OUT_EXTRA = N - NS * RPT_O


def _sc_segment_sum(x, src4, dst4):
  mesh = plsc.VectorSubcoreMesh(core_axis_name="c", subcore_axis_name="s")

  @functools.partial(
      pl.kernel,
      out_type=jax.ShapeDtypeStruct((NC, N, D), jnp.float32),
      mesh=mesh,
      scratch_types=[
          pltpu.VMEM((GT, B), jnp.int32),
          pltpu.VMEM((GT, B), jnp.int32),
          pltpu.VMEM((GT, B), jnp.int32),
          pltpu.VMEM((GT, B), jnp.int32),
          pltpu.VMEM((B, D), jnp.float32),
          pltpu.VMEM((B, D), jnp.float32),
          pltpu.VMEM_SHARED((NROWS, D), jnp.float32),
          pltpu.SemaphoreType.DMA,
          pltpu.SemaphoreType.DMA,
          pltpu.SemaphoreType.DMA,
          pltpu.SemaphoreType.DMA,
      ],
  )
  def k(x_hbm, src_hbm, dst_hbm, out_hbm, sidx0, didx0, sidx1, didx1,
        rows0, rows1, agg, sem0, sem1, isem0, isem1):
    c = lax.axis_index("c")
    s = lax.axis_index("s")
    t = c * NS + s

    def stage(g, sidx, didx, isem):
      pltpu.async_copy(src_hbm.at[t, g], sidx, isem)
      pltpu.async_copy(dst_hbm.at[t, g], didx, isem)

    def swait(sidx, didx, isem):
      pltpu.make_async_copy(src_hbm.at[t, 0], sidx, isem).wait()
      pltpu.make_async_copy(dst_hbm.at[t, 0], didx, isem).wait()

    def gather(j, rows, sem, sidx):
      pltpu.async_copy(x_hbm.at[sidx.at[j]], rows, sem)

    def gwait(rows, sem, sidx):
      pltpu.make_async_copy(x_hbm.at[sidx.at[0]], rows, sem).wait()

    def scat(j, rows, didx):
      pltpu.sync_copy(rows, agg.at[didx.at[j]], add=True)

    stage(0, sidx0, didx0, isem0)
    stage(1, sidx1, didx1, isem1)

    zv = jnp.zeros((16,), jnp.float32)

    def zbody(i, c2):
      for l in range(D // 16):
        rows0[i, pl.ds(l * 16, 16)] = zv
      return c2

    lax.fori_loop(0, B, zbody, 0)
    zbase = s * RPT_Z
    for kk in range(RPT_Z // B):
      pltpu.async_copy(rows0, agg.at[pl.ds(zbase + kk * B, B)], sem0)
    zrem = RPT_Z - (RPT_Z // B) * B
    pltpu.async_copy(rows0.at[pl.ds(0, zrem)],
                     agg.at[pl.ds(zbase + (RPT_Z // B) * B, zrem)], sem1)
    for kk in range(RPT_Z // B):
      pltpu.make_async_copy(rows0, agg.at[pl.ds(zbase + kk * B, B)],
                            sem0).wait()
    pltpu.make_async_copy(rows0.at[pl.ds(0, zrem)],
                          agg.at[pl.ds(zbase + (RPT_Z // B) * B, zrem)],
                          sem1).wait()
    plsc.subcore_barrier()

    def rungroup(sidx, didx):
      gather(0, rows0, sem0, sidx)

      def body(i, c2):
        j0 = 2 * i
        gather(j0 + 1, rows1, sem1, sidx)
        gwait(rows0, sem0, sidx)
        scat(j0, rows0, didx)

        @pl.when(j0 + 2 < GT)
        def _():
          gather(j0 + 2, rows0, sem0, sidx)

        gwait(rows1, sem1, sidx)
        scat(j0 + 1, rows1, didx)
        return c2

      lax.fori_loop(0, GT // 2, body, 0)

    def pair(p_, carry):
      g0 = 2 * p_
      swait(sidx0, didx0, isem0)
      rungroup(sidx0, didx0)

      @pl.when(g0 + 2 < NG)
      def _():
        stage(g0 + 2, sidx0, didx0, isem0)

      swait(sidx1, didx1, isem1)
      rungroup(sidx1, didx1)

      @pl.when(g0 + 3 < NG)
      def _():
        stage(g0 + 3, sidx1, didx1, isem1)

      return carry

    lax.fori_loop(0, NG // 2, pair, 0)
    plsc.subcore_barrier()
    pltpu.sync_copy(agg.at[pl.ds(s * RPT_O, RPT_O)],
                    out_hbm.at[c, pl.ds(s * RPT_O, RPT_O)])

    @pl.when(s == NS - 1)
    def _():
      pltpu.sync_copy(agg.at[pl.ds(NS * RPT_O, OUT_EXTRA)],
                      out_hbm.at[c, pl.ds(NS * RPT_O, OUT_EXTRA)])

  return k(x, src4, dst4)


_SBN = float(1.0 / (1.0 + BN_EPS) ** 0.5)


def _mlp(eps2, x, p, W1, b1, g1, be1, W2, b2, g2, be2):
  RB = 1000
  G = N // RB

  def body(eps_ref, x_ref, p0_ref, p1_ref, w1_ref, b1_ref, g1_ref, be1_ref,
           w2_ref, b2_ref, g2_ref, be2_ref, o_ref):
    xb = x_ref[...]
    a = xb * (1.0 + eps_ref[0, 0]) + p0_ref[0] + p1_ref[0]
    w1 = (w1_ref[...] * (g1_ref[...] * _SBN)).astype(jnp.bfloat16)
    h = jnp.dot(a.astype(jnp.bfloat16), w1,
                preferred_element_type=jnp.float32)
    h = jnp.maximum(h + (b1_ref[...] * g1_ref[...] * _SBN + be1_ref[...]),
                    0.0)
    w2 = (w2_ref[...] * (g2_ref[...] * _SBN)).astype(jnp.bfloat16)
    o = jnp.dot(h.astype(jnp.bfloat16), w2,
                preferred_element_type=jnp.float32)
    o_ref[...] = jnp.maximum(
        o + (b2_ref[...] * g2_ref[...] * _SBN + be2_ref[...]) + xb, 0.0)

  row = lambda i: (0, 0)
  return pl.pallas_call(
      body,
      grid=(G,),
      in_specs=[
          pl.BlockSpec(memory_space=pltpu.SMEM),
          pl.BlockSpec((RB, D), lambda i: (i, 0)),
          pl.BlockSpec((1, RB, D), lambda i: (0, i, 0)),
          pl.BlockSpec((1, RB, D), lambda i: (1, i, 0)),
          pl.BlockSpec((D, H), row),
          pl.BlockSpec((1, H), row),
          pl.BlockSpec((1, H), row),
          pl.BlockSpec((1, H), row),
          pl.BlockSpec((H, D), row),
          pl.BlockSpec((1, D), row),
          pl.BlockSpec((1, D), row),
          pl.BlockSpec((1, D), row),
      ],
      out_specs=pl.BlockSpec((RB, D), lambda i: (i, 0)),
      out_shape=jax.ShapeDtypeStruct((N, D), jnp.float32),
  )(eps2, x, p, p, W1, b1.reshape(1, H), g1.reshape(1, H),
    be1.reshape(1, H), W2, b2.reshape(1, D), g2.reshape(1, D),
    be2.reshape(1, D))


def kernel(x, edge_index, eps, W1, b1, g1, be1, W2, b2, g2, be2):
  src = edge_index[0]
  dst = edge_index[1]
  ar = jnp.arange(PAD, dtype=jnp.int32)
  src4 = jnp.concatenate([src, ar % N]).reshape(NT, NG, GT, B)
  dst4 = jnp.concatenate([dst, N + (ar % NPADROWS)]).reshape(NT, NG, GT, B)
  p = _sc_segment_sum(x, src4, dst4)
  return _mlp(eps.reshape(1, 1), x, p, W1, b1, g1, be1, W2, b2, g2, be2)


# speedup vs baseline: 1.0813x; 1.0813x over previous
import functools

import jax
import jax.numpy as jnp
from jax import lax
from jax.experimental import pallas as pl
from jax.experimental.pallas import tpu as pltpu
from jax.experimental.pallas import tpu_sc as plsc

N = 10000
D = 128
H = 256
E = 320000
BN_EPS = 1e-5

NC = 2
NS = 16
NT = NC * NS
B = 128
GT = 20
NG = 4
NGR = E // (GT * B)
NROWS = N + 112
RPT_Z = NROWS // NS
RPT_O = 624
OUT_EXTRA = N - NS * RPT_O


def _sc_segment_sum(x, eidx4):
  mesh = plsc.VectorSubcoreMesh(core_axis_name="c", subcore_axis_name="s")

  @functools.partial(
      pl.kernel,
      out_type=jax.ShapeDtypeStruct((NC, N, D), jnp.float32),
      mesh=mesh,
      scratch_types=[
          pltpu.VMEM((GT, B), jnp.int32),
          pltpu.VMEM((GT, B), jnp.int32),
          pltpu.VMEM((GT, B), jnp.int32),
          pltpu.VMEM((GT, B), jnp.int32),
          pltpu.VMEM((B, D), jnp.float32),
          pltpu.VMEM((B, D), jnp.float32),
          pltpu.VMEM_SHARED((NROWS, D), jnp.float32),
          pltpu.SemaphoreType.DMA,
          pltpu.SemaphoreType.DMA,
          pltpu.SemaphoreType.DMA,
          pltpu.SemaphoreType.DMA,
      ],
  )
  def k(x_hbm, eidx_hbm, out_hbm, sidx0, didx0, sidx1, didx1,
        rows0, rows1, agg, sem0, sem1, isem0, isem1):
    c = lax.axis_index("c")
    s = lax.axis_index("s")
    t = c * NS + s
    fgbase = t * NG

    def stage(fg, sidx, didx, isem):
      pltpu.async_copy(eidx_hbm.at[0, fg], sidx, isem)
      pltpu.async_copy(eidx_hbm.at[1, fg], didx, isem)

    def swait(sidx, didx, isem):
      pltpu.make_async_copy(eidx_hbm.at[0, 0], sidx, isem).wait()
      pltpu.make_async_copy(eidx_hbm.at[1, 0], didx, isem).wait()

    def gather(j, rows, sem, sidx):
      pltpu.async_copy(x_hbm.at[sidx.at[j]], rows, sem)

    def gwait(rows, sem, sidx):
      pltpu.make_async_copy(x_hbm.at[sidx.at[0]], rows, sem).wait()

    def scat(j, rows, didx):
      pltpu.sync_copy(rows, agg.at[didx.at[j]], add=True)

    stage(fgbase, sidx0, didx0, isem0)

    @pl.when(fgbase + 1 < NGR)
    def _():
      stage(fgbase + 1, sidx1, didx1, isem1)

    zv = jnp.zeros((16,), jnp.float32)

    def zbody(i, c2):
      for l in range(D // 16):
        rows0[i, pl.ds(l * 16, 16)] = zv
      return c2

    lax.fori_loop(0, B, zbody, 0)
    zbase = s * RPT_Z
    for kk in range(RPT_Z // B):
      pltpu.async_copy(rows0, agg.at[pl.ds(zbase + kk * B, B)], sem0)
    zrem = RPT_Z - (RPT_Z // B) * B
    pltpu.async_copy(rows0.at[pl.ds(0, zrem)],
                     agg.at[pl.ds(zbase + (RPT_Z // B) * B, zrem)], sem1)
    for kk in range(RPT_Z // B):
      pltpu.make_async_copy(rows0, agg.at[pl.ds(zbase + kk * B, B)],
                            sem0).wait()
    pltpu.make_async_copy(rows0.at[pl.ds(0, zrem)],
                          agg.at[pl.ds(zbase + (RPT_Z // B) * B, zrem)],
                          sem1).wait()
    plsc.subcore_barrier()

    def rungroup(sidx, didx):
      gather(0, rows0, sem0, sidx)

      def body(i, c2):
        j0 = 2 * i
        gather(j0 + 1, rows1, sem1, sidx)
        gwait(rows0, sem0, sidx)
        scat(j0, rows0, didx)

        @pl.when(j0 + 2 < GT)
        def _():
          gather(j0 + 2, rows0, sem0, sidx)

        gwait(rows1, sem1, sidx)
        scat(j0 + 1, rows1, didx)
        return c2

      lax.fori_loop(0, GT // 2, body, 0)

    def pair(p_, carry):
      g0 = 2 * p_
      fg0 = fgbase + g0

      @pl.when(fg0 < NGR)
      def _():
        swait(sidx0, didx0, isem0)
        rungroup(sidx0, didx0)

      @pl.when(jnp.logical_and(g0 + 2 < NG, fg0 + 2 < NGR))
      def _():
        stage(fg0 + 2, sidx0, didx0, isem0)

      @pl.when(fg0 + 1 < NGR)
      def _():
        swait(sidx1, didx1, isem1)
        rungroup(sidx1, didx1)

      @pl.when(jnp.logical_and(g0 + 3 < NG, fg0 + 3 < NGR))
      def _():
        stage(fg0 + 3, sidx1, didx1, isem1)

      return carry

    lax.fori_loop(0, NG // 2, pair, 0)
    plsc.subcore_barrier()
    pltpu.sync_copy(agg.at[pl.ds(s * RPT_O, RPT_O)],
                    out_hbm.at[c, pl.ds(s * RPT_O, RPT_O)])

    @pl.when(s == NS - 1)
    def _():
      pltpu.sync_copy(agg.at[pl.ds(NS * RPT_O, OUT_EXTRA)],
                      out_hbm.at[c, pl.ds(NS * RPT_O, OUT_EXTRA)])

  return k(x, eidx4)


_SBN = float(1.0 / (1.0 + BN_EPS) ** 0.5)


def _mlp(eps2, x, p, W1, b1, g1, be1, W2, b2, g2, be2):
  RB = 1000
  G = N // RB

  def body(eps_ref, x_ref, p0_ref, p1_ref, w1_ref, b1_ref, g1_ref, be1_ref,
           w2_ref, b2_ref, g2_ref, be2_ref, o_ref):
    xb = x_ref[...]
    a = xb * (1.0 + eps_ref[0, 0]) + p0_ref[0] + p1_ref[0]
    w1 = (w1_ref[...] * (g1_ref[...] * _SBN)).astype(jnp.bfloat16)
    h = jnp.dot(a.astype(jnp.bfloat16), w1,
                preferred_element_type=jnp.float32)
    h = jnp.maximum(h + (b1_ref[...] * g1_ref[...] * _SBN + be1_ref[...]),
                    0.0)
    w2 = (w2_ref[...] * (g2_ref[...] * _SBN)).astype(jnp.bfloat16)
    o = jnp.dot(h.astype(jnp.bfloat16), w2,
                preferred_element_type=jnp.float32)
    o_ref[...] = jnp.maximum(
        o + (b2_ref[...] * g2_ref[...] * _SBN + be2_ref[...]) + xb, 0.0)

  row = lambda i: (0, 0)
  return pl.pallas_call(
      body,
      grid=(G,),
      in_specs=[
          pl.BlockSpec(memory_space=pltpu.SMEM),
          pl.BlockSpec((RB, D), lambda i: (i, 0)),
          pl.BlockSpec((1, RB, D), lambda i: (0, i, 0)),
          pl.BlockSpec((1, RB, D), lambda i: (1, i, 0)),
          pl.BlockSpec((D, H), row),
          pl.BlockSpec((1, H), row),
          pl.BlockSpec((1, H), row),
          pl.BlockSpec((1, H), row),
          pl.BlockSpec((H, D), row),
          pl.BlockSpec((1, D), row),
          pl.BlockSpec((1, D), row),
          pl.BlockSpec((1, D), row),
      ],
      out_specs=pl.BlockSpec((RB, D), lambda i: (i, 0)),
      out_shape=jax.ShapeDtypeStruct((N, D), jnp.float32),
  )(eps2, x, p, p, W1, b1.reshape(1, H), g1.reshape(1, H),
    be1.reshape(1, H), W2, b2.reshape(1, D), g2.reshape(1, D),
    be2.reshape(1, D))


def kernel(x, edge_index, eps, W1, b1, g1, be1, W2, b2, g2, be2):
  eidx4 = edge_index.reshape(2, NGR, GT, B)
  p = _sc_segment_sum(x, eidx4)
  return _mlp(eps.reshape(1, 1), x, p, W1, b1, g1, be1, W2, b2, g2, be2)


# speedup vs baseline: 1.0938x; 1.0116x over previous
import functools

import jax
import jax.numpy as jnp
from jax import lax
from jax.experimental import pallas as pl
from jax.experimental.pallas import tpu as pltpu
from jax.experimental.pallas import tpu_sc as plsc

N = 10000
D = 128
H = 256
E = 320000
BN_EPS = 1e-5

NC = 2
NS = 16
NT = NC * NS
B = 128
GT = 20
NG = 4
NGR = E // (GT * B)
NROWS = N + 112
RPT_Z = NROWS // NS
RPT_O = 624
OUT_EXTRA = N - NS * RPT_O


def _sc_segment_sum(x, eidx4):
  mesh = plsc.VectorSubcoreMesh(core_axis_name="c", subcore_axis_name="s")

  @functools.partial(
      pl.kernel,
      out_type=jax.ShapeDtypeStruct((NC, N, D), jnp.float32),
      mesh=mesh,
      scratch_types=[
          pltpu.VMEM((GT, B), jnp.int32),
          pltpu.VMEM((GT, B), jnp.int32),
          pltpu.VMEM((GT, B), jnp.int32),
          pltpu.VMEM((GT, B), jnp.int32),
          pltpu.VMEM((B, D), jnp.float32),
          pltpu.VMEM((B, D), jnp.float32),
          pltpu.VMEM_SHARED((NROWS, D), jnp.float32),
          pltpu.SemaphoreType.DMA,
          pltpu.SemaphoreType.DMA,
          pltpu.SemaphoreType.DMA,
          pltpu.SemaphoreType.DMA,
      ],
  )
  def k(x_hbm, eidx_hbm, out_hbm, sidx0, didx0, sidx1, didx1,
        rows0, rows1, agg, sem0, sem1, isem0, isem1):
    c = lax.axis_index("c")
    s = lax.axis_index("s")
    t = c * NS + s
    fgbase = t * NG

    def stage(fg, sidx, didx, isem):
      pltpu.async_copy(eidx_hbm.at[0, fg], sidx, isem)
      pltpu.async_copy(eidx_hbm.at[1, fg], didx, isem)

    def swait(sidx, didx, isem):
      pltpu.make_async_copy(eidx_hbm.at[0, 0], sidx, isem).wait()
      pltpu.make_async_copy(eidx_hbm.at[1, 0], didx, isem).wait()

    def gather(j, rows, sem, sidx):
      pltpu.async_copy(x_hbm.at[sidx.at[j]], rows, sem)

    def gwait(rows, sem, sidx):
      pltpu.make_async_copy(x_hbm.at[sidx.at[0]], rows, sem).wait()

    def scat(j, rows, didx):
      pltpu.sync_copy(rows, agg.at[didx.at[j]], add=True)

    stage(fgbase, sidx0, didx0, isem0)

    @pl.when(fgbase + 1 < NGR)
    def _():
      stage(fgbase + 1, sidx1, didx1, isem1)

    zv = jnp.zeros((16,), jnp.float32)

    def zbody(i, c2):
      for l in range(D // 16):
        rows0[i, pl.ds(l * 16, 16)] = zv
      return c2

    lax.fori_loop(0, B, zbody, 0)
    zbase = s * RPT_Z
    for kk in range(RPT_Z // B):
      pltpu.async_copy(rows0, agg.at[pl.ds(zbase + kk * B, B)], sem0)
    zrem = RPT_Z - (RPT_Z // B) * B
    pltpu.async_copy(rows0.at[pl.ds(0, zrem)],
                     agg.at[pl.ds(zbase + (RPT_Z // B) * B, zrem)], sem1)
    for kk in range(RPT_Z // B):
      pltpu.make_async_copy(rows0, agg.at[pl.ds(zbase + kk * B, B)],
                            sem0).wait()
    pltpu.make_async_copy(rows0.at[pl.ds(0, zrem)],
                          agg.at[pl.ds(zbase + (RPT_Z // B) * B, zrem)],
                          sem1).wait()
    plsc.subcore_barrier()

    def rungroup(sidx, didx):
      gather(0, rows0, sem0, sidx)

      def body(i, c2):
        j0 = 2 * i
        gather(j0 + 1, rows1, sem1, sidx)
        gwait(rows0, sem0, sidx)
        scat(j0, rows0, didx)

        @pl.when(j0 + 2 < GT)
        def _():
          gather(j0 + 2, rows0, sem0, sidx)

        gwait(rows1, sem1, sidx)
        scat(j0 + 1, rows1, didx)
        return c2

      lax.fori_loop(0, GT // 2, body, 0)

    def pair(p_, carry):
      g0 = 2 * p_
      fg0 = fgbase + g0

      @pl.when(fg0 < NGR)
      def _():
        swait(sidx0, didx0, isem0)
        rungroup(sidx0, didx0)

      @pl.when(jnp.logical_and(g0 + 2 < NG, fg0 + 2 < NGR))
      def _():
        stage(fg0 + 2, sidx0, didx0, isem0)

      @pl.when(fg0 + 1 < NGR)
      def _():
        swait(sidx1, didx1, isem1)
        rungroup(sidx1, didx1)

      @pl.when(jnp.logical_and(g0 + 3 < NG, fg0 + 3 < NGR))
      def _():
        stage(fg0 + 3, sidx1, didx1, isem1)

      return carry

    lax.fori_loop(0, NG // 2, pair, 0)
    plsc.subcore_barrier()
    pltpu.sync_copy(agg.at[pl.ds(s * RPT_O, RPT_O)],
                    out_hbm.at[c, pl.ds(s * RPT_O, RPT_O)])

    @pl.when(s == NS - 1)
    def _():
      pltpu.sync_copy(agg.at[pl.ds(NS * RPT_O, OUT_EXTRA)],
                      out_hbm.at[c, pl.ds(NS * RPT_O, OUT_EXTRA)])

  return k(x, eidx4)


_SBN = float(1.0 / (1.0 + BN_EPS) ** 0.5)


def _mlp(eps2, x, p, W1, b1, g1, be1, W2, b2, g2, be2):
  RB = 2000
  G = N // RB

  def body(eps_ref, x_ref, p0_ref, p1_ref, w1_ref, b1_ref, g1_ref, be1_ref,
           w2_ref, b2_ref, g2_ref, be2_ref, o_ref):
    xb = x_ref[...]
    a = xb * (1.0 + eps_ref[0, 0]) + p0_ref[0] + p1_ref[0]
    w1 = (w1_ref[...] * (g1_ref[...] * _SBN)).astype(jnp.bfloat16)
    h = jnp.dot(a.astype(jnp.bfloat16), w1,
                preferred_element_type=jnp.float32)
    h = jnp.maximum(h + (b1_ref[...] * g1_ref[...] * _SBN + be1_ref[...]),
                    0.0)
    w2 = (w2_ref[...] * (g2_ref[...] * _SBN)).astype(jnp.bfloat16)
    o = jnp.dot(h.astype(jnp.bfloat16), w2,
                preferred_element_type=jnp.float32)
    o_ref[...] = jnp.maximum(
        o + (b2_ref[...] * g2_ref[...] * _SBN + be2_ref[...]) + xb, 0.0)

  row = lambda i: (0, 0)
  return pl.pallas_call(
      body,
      grid=(G,),
      in_specs=[
          pl.BlockSpec(memory_space=pltpu.SMEM),
          pl.BlockSpec((RB, D), lambda i: (i, 0)),
          pl.BlockSpec((1, RB, D), lambda i: (0, i, 0)),
          pl.BlockSpec((1, RB, D), lambda i: (1, i, 0)),
          pl.BlockSpec((D, H), row),
          pl.BlockSpec((1, H), row),
          pl.BlockSpec((1, H), row),
          pl.BlockSpec((1, H), row),
          pl.BlockSpec((H, D), row),
          pl.BlockSpec((1, D), row),
          pl.BlockSpec((1, D), row),
          pl.BlockSpec((1, D), row),
      ],
      out_specs=pl.BlockSpec((RB, D), lambda i: (i, 0)),
      out_shape=jax.ShapeDtypeStruct((N, D), jnp.float32),
  )(eps2, x, p, p, W1, b1.reshape(1, H), g1.reshape(1, H),
    be1.reshape(1, H), W2, b2.reshape(1, D), g2.reshape(1, D),
    be2.reshape(1, D))


def kernel(x, edge_index, eps, W1, b1, g1, be1, W2, b2, g2, be2):
  eidx4 = edge_index.reshape(2, NGR, GT, B)
  p = _sc_segment_sum(x, eidx4)
  return _mlp(eps.reshape(1, 1), x, p, W1, b1, g1, be1, W2, b2, g2, be2)


# speedup vs baseline: 1.1215x; 1.0253x over previous
import functools

import jax
import jax.numpy as jnp
from jax import lax
from jax.experimental import pallas as pl
from jax.experimental.pallas import tpu as pltpu
from jax.experimental.pallas import tpu_sc as plsc

N = 10000
D = 128
H = 256
E = 320000
BN_EPS = 1e-5

NC = 2
NS = 16
NT = NC * NS
B = 128
GT = 20
NG = 4
NGR = E // (GT * B)
NROWS = N + 112
RPT_Z = NROWS // NS
RPT_O = 624
OUT_EXTRA = N - NS * RPT_O


def _sc_segment_sum(x, eidx4):
  mesh = plsc.VectorSubcoreMesh(core_axis_name="c", subcore_axis_name="s")

  @functools.partial(
      pl.kernel,
      out_type=jax.ShapeDtypeStruct((NC, N, D), jnp.float32),
      mesh=mesh,
      scratch_types=[
          pltpu.VMEM((GT, B), jnp.int32),
          pltpu.VMEM((GT, B), jnp.int32),
          pltpu.VMEM((GT, B), jnp.int32),
          pltpu.VMEM((GT, B), jnp.int32),
          pltpu.VMEM((B, D), jnp.float32),
          pltpu.VMEM((B, D), jnp.float32),
          pltpu.VMEM_SHARED((NROWS, D), jnp.float32),
          pltpu.SemaphoreType.DMA,
          pltpu.SemaphoreType.DMA,
          pltpu.SemaphoreType.DMA,
          pltpu.SemaphoreType.DMA,
      ],
  )
  def k(x_hbm, eidx_hbm, out_hbm, sidx0, didx0, sidx1, didx1,
        rows0, rows1, agg, sem0, sem1, isem0, isem1):
    c = lax.axis_index("c")
    s = lax.axis_index("s")
    t = c * NS + s
    fgbase = t * NG

    def stage(fg, sidx, didx, isem):
      pltpu.async_copy(eidx_hbm.at[0, fg], sidx, isem)
      pltpu.async_copy(eidx_hbm.at[1, fg], didx, isem)

    def swait(sidx, didx, isem):
      pltpu.make_async_copy(eidx_hbm.at[0, 0], sidx, isem).wait()
      pltpu.make_async_copy(eidx_hbm.at[1, 0], didx, isem).wait()

    def gather(j, rows, sem, sidx):
      pltpu.async_copy(x_hbm.at[sidx.at[j]], rows, sem)

    def gwait(rows, sem, sidx):
      pltpu.make_async_copy(x_hbm.at[sidx.at[0]], rows, sem).wait()

    def scat(j, rows, didx):
      pltpu.sync_copy(rows, agg.at[didx.at[j]], add=True)

    stage(fgbase, sidx0, didx0, isem0)

    @pl.when(fgbase + 1 < NGR)
    def _():
      stage(fgbase + 1, sidx1, didx1, isem1)

    swait(sidx0, didx0, isem0)

    zv = jnp.zeros((16,), jnp.float32)

    def zbody(i, c2):
      for l in range(D // 16):
        rows0[i, pl.ds(l * 16, 16)] = zv
      return c2

    lax.fori_loop(0, B, zbody, 0)
    zbase = s * RPT_Z
    for kk in range(RPT_Z // B):
      pltpu.async_copy(rows0, agg.at[pl.ds(zbase + kk * B, B)], sem0)
    zrem = RPT_Z - (RPT_Z // B) * B
    pltpu.async_copy(rows0.at[pl.ds(0, zrem)],
                     agg.at[pl.ds(zbase + (RPT_Z // B) * B, zrem)], sem1)
    for kk in range(RPT_Z // B):
      pltpu.make_async_copy(rows0, agg.at[pl.ds(zbase + kk * B, B)],
                            sem0).wait()
    pltpu.make_async_copy(rows0.at[pl.ds(0, zrem)],
                          agg.at[pl.ds(zbase + (RPT_Z // B) * B, zrem)],
                          sem1).wait()
    plsc.subcore_barrier()
    gather(0, rows0, sem0, sidx0)

    def rungroup(sidx, didx, nxt_pred, nxt_sidx, nxt_didx, nxt_isem):
      def body(i, c2):
        j0 = 2 * i
        gather(j0 + 1, rows1, sem1, sidx)
        gwait(rows0, sem0, sidx)
        scat(j0, rows0, didx)
        gather(j0 + 2, rows0, sem0, sidx)
        gwait(rows1, sem1, sidx)
        scat(j0 + 1, rows1, didx)
        return c2

      lax.fori_loop(0, GT // 2 - 1, body, 0)
      j0 = GT - 2
      gather(j0 + 1, rows1, sem1, sidx)
      gwait(rows0, sem0, sidx)
      scat(j0, rows0, didx)

      @pl.when(nxt_pred)
      def _():
        swait(nxt_sidx, nxt_didx, nxt_isem)
        gather(0, rows0, sem0, nxt_sidx)

      gwait(rows1, sem1, sidx)
      scat(j0 + 1, rows1, didx)

    def pair(p_, carry):
      g0 = 2 * p_
      fg0 = fgbase + g0

      @pl.when(fg0 < NGR)
      def _():
        rungroup(sidx0, didx0, fg0 + 1 < NGR, sidx1, didx1, isem1)

      @pl.when(jnp.logical_and(g0 + 2 < NG, fg0 + 2 < NGR))
      def _():
        stage(fg0 + 2, sidx0, didx0, isem0)

      @pl.when(fg0 + 1 < NGR)
      def _():
        rungroup(sidx1, didx1,
                 jnp.logical_and(g0 + 2 < NG, fg0 + 2 < NGR),
                 sidx0, didx0, isem0)

      @pl.when(jnp.logical_and(g0 + 3 < NG, fg0 + 3 < NGR))
      def _():
        stage(fg0 + 3, sidx1, didx1, isem1)

      return carry

    lax.fori_loop(0, NG // 2, pair, 0)
    plsc.subcore_barrier()
    pltpu.sync_copy(agg.at[pl.ds(s * RPT_O, RPT_O)],
                    out_hbm.at[c, pl.ds(s * RPT_O, RPT_O)])

    @pl.when(s == NS - 1)
    def _():
      pltpu.sync_copy(agg.at[pl.ds(NS * RPT_O, OUT_EXTRA)],
                      out_hbm.at[c, pl.ds(NS * RPT_O, OUT_EXTRA)])

  return k(x, eidx4)


_SBN = float(1.0 / (1.0 + BN_EPS) ** 0.5)


def _mlp(eps2, x, p, W1, b1, g1, be1, W2, b2, g2, be2):
  RB = 2000
  G = N // RB

  def body(eps_ref, x_ref, p0_ref, p1_ref, w1_ref, b1_ref, g1_ref, be1_ref,
           w2_ref, b2_ref, g2_ref, be2_ref, o_ref):
    xb = x_ref[...]
    a = xb * (1.0 + eps_ref[0, 0]) + p0_ref[0] + p1_ref[0]
    w1 = (w1_ref[...] * (g1_ref[...] * _SBN)).astype(jnp.bfloat16)
    h = jnp.dot(a.astype(jnp.bfloat16), w1,
                preferred_element_type=jnp.float32)
    h = jnp.maximum(h + (b1_ref[...] * g1_ref[...] * _SBN + be1_ref[...]),
                    0.0)
    w2 = (w2_ref[...] * (g2_ref[...] * _SBN)).astype(jnp.bfloat16)
    o = jnp.dot(h.astype(jnp.bfloat16), w2,
                preferred_element_type=jnp.float32)
    o_ref[...] = jnp.maximum(
        o + (b2_ref[...] * g2_ref[...] * _SBN + be2_ref[...]) + xb, 0.0)

  row = lambda i: (0, 0)
  return pl.pallas_call(
      body,
      grid=(G,),
      in_specs=[
          pl.BlockSpec(memory_space=pltpu.SMEM),
          pl.BlockSpec((RB, D), lambda i: (i, 0)),
          pl.BlockSpec((1, RB, D), lambda i: (0, i, 0)),
          pl.BlockSpec((1, RB, D), lambda i: (1, i, 0)),
          pl.BlockSpec((D, H), row),
          pl.BlockSpec((1, H), row),
          pl.BlockSpec((1, H), row),
          pl.BlockSpec((1, H), row),
          pl.BlockSpec((H, D), row),
          pl.BlockSpec((1, D), row),
          pl.BlockSpec((1, D), row),
          pl.BlockSpec((1, D), row),
      ],
      out_specs=pl.BlockSpec((RB, D), lambda i: (i, 0)),
      out_shape=jax.ShapeDtypeStruct((N, D), jnp.float32),
  )(eps2, x, p, p, W1, b1.reshape(1, H), g1.reshape(1, H),
    be1.reshape(1, H), W2, b2.reshape(1, D), g2.reshape(1, D),
    be2.reshape(1, D))


def kernel(x, edge_index, eps, W1, b1, g1, be1, W2, b2, g2, be2):
  eidx4 = edge_index.reshape(2, NGR, GT, B)
  p = _sc_segment_sum(x, eidx4)
  return _mlp(eps.reshape(1, 1), x, p, W1, b1, g1, be1, W2, b2, g2, be2)
